# Initial kernel scaffold; baseline (speedup 1.0000x reference)
import jax
import jax.numpy as jnp
from jax.experimental import pallas as pl


def kernel(hidden_states, h0, c0, Wk, bk, Wv, bv, Wq, Wi, Wh, Wq2, Wk2, Wv2, Wout):
    raise NotImplementedError("write your pallas kernel here")


# speedup vs baseline: 3.1484x; 3.1484x over previous
import math

import jax
import jax.numpy as jnp
from jax.experimental import pallas as pl
from jax.experimental.pallas import tpu as pltpu

B, T, D = 1, 2048, 1024
E, K, H = 8, 2, 128
IKS, IVS = 64, 400
NCH, CKS, CVS = 4, 32, 128
G4 = 4 * H

CHUNK = 256


def _stage2(kk_ref, vv_ref, h0_ref, c0_ref, wqf_ref, wif_ref,
            whf_ref, wqkv_ref, wout3_ref, rw_ref, idx_ref, h_s, c_s):
    @pl.when(pl.program_id(0) == 0)
    def _():
        h_s[...] = h0_ref[0]
        c_s[...] = c0_ref[0]

    uidx = jax.lax.broadcasted_iota(jnp.int32, (E, 1), 0)
    lane = jax.lax.broadcasted_iota(jnp.int32, (1, 128), 1)

    wqf = wqf_ref[...]
    wif = wif_ref[...]
    whf = whf_ref[...]
    wqkv = wqkv_ref[...]
    wout3 = wout3_ref[...]

    def blkdiag(R, G):
        acc = jnp.where(uidx == 0, R[:, 0:G], 0.0)
        for j in range(1, E):
            acc = acc + jnp.where(uidx == j, R[:, j * G:(j + 1) * G], 0.0)
        return acc

    dd = lambda a, b: jnp.dot(a, b, preferred_element_type=jnp.float32)

    def step(t, carry):
        h, c = carry
        kk = kk_ref[t]
        vv = vv_ref[t]
        q = blkdiag(dd(h, wqf), IKS)
        sc = jax.lax.dot_general(q, kk, (((1,), (1,)), ((), ())),
                                 preferred_element_type=jnp.float32) / math.sqrt(IKS)
        s0 = sc[:, 0:1]
        m1 = jnp.max(s0)
        i1 = jnp.min(jnp.where(s0 >= m1, uidx, E))
        s0m = jnp.where(uidx == i1, -jnp.inf, s0)
        m2 = jnp.max(s0m)
        i2 = jnp.min(jnp.where(s0m >= m2, uidx, E))
        maskf = jnp.logical_or(uidx == i1, uidx == i2).astype(jnp.float32)
        mx = jnp.max(sc, axis=1, keepdims=True)
        ex = jnp.exp(sc - mx)
        probs = ex / jnp.sum(ex, axis=1, keepdims=True)
        j1 = jnp.minimum(i1, i2)
        j2 = jnp.maximum(i1, i2)
        p0 = probs[:, 0:1]
        w1 = jnp.sum(jnp.where(uidx == j1, p0, 0.0))
        w2 = jnp.sum(jnp.where(uidx == j2, p0, 0.0))
        wsum = w1 + w2
        inp = dd(probs, vv) * maskf
        preact = blkdiag(dd(inp, wif), G4) + blkdiag(dd(h, whf), G4)
        i_g = jax.nn.sigmoid(preact[:, 0:H])
        f_g = jax.nn.sigmoid(preact[:, H:2 * H])
        o_g = jax.nn.sigmoid(preact[:, 2 * H:3 * H])
        g_g = jnp.tanh(preact[:, 3 * H:4 * H])
        c_new = c * f_g + i_g * g_g
        h_new = o_g * jnp.tanh(c_new)
        R2 = dd(h_new, wqkv)
        hq = blkdiag(R2[:, 0:1024], NCH * CKS)
        hk = blkdiag(R2[:, 1024:2048], NCH * CKS)
        hv = blkdiag(R2[:, 2048:6144], NCH * CVS)
        ctx_heads = []
        for n in range(NCH):
            qn = hq[:, n * CKS:(n + 1) * CKS]
            kn = hk[:, n * CKS:(n + 1) * CKS]
            vn = hv[:, n * CVS:(n + 1) * CVS]
            att = jax.lax.dot_general(
                qn, kn, (((1,), (1,)), ((), ())),
                preferred_element_type=jnp.float32) / math.sqrt(CKS)
            amx = jnp.max(att, axis=1, keepdims=True)
            aex = jnp.exp(att - amx)
            att = aex / jnp.sum(aex, axis=1, keepdims=True)
            ctx_heads.append(dd(att, vn))
        ctx = jnp.concatenate(ctx_heads, axis=1)
        ctxo = jax.lax.dot_general(
            ctx[:, None, :], wout3, (((2,), (1,)), ((0,), (0,))),
            preferred_element_type=jnp.float32)[:, 0, :] + h_new
        h_n = maskf * ctxo + (1.0 - maskf) * h
        c_n = maskf * c_new + (1.0 - maskf) * c
        rw_ref[pl.ds(t, 1), :] = jnp.where(
            lane == 0, w1 / wsum, jnp.where(lane == 1, w2 / wsum, 0.0))
        idx_ref[pl.ds(t, 1), :] = jnp.where(
            lane == 0, j1, jnp.where(lane == 1, j2, 0))
        return (h_n, c_n)

    h_f, c_f = jax.lax.fori_loop(0, CHUNK, step, (h_s[...], c_s[...]))
    h_s[...] = h_f
    c_s[...] = c_f


def kernel(hidden_states, h0, c0, Wk, bk, Wv, bv, Wq, Wi, Wh, Wq2, Wk2, Wv2, Wout):
    xs = jnp.transpose(hidden_states, (1, 0, 2))

    def proj(carry, x_t):
        x = x_t[:, None, :]
        xx = jnp.concatenate([x, jnp.zeros_like(x)], axis=1)
        kk = xx @ Wk + bk
        vv = xx @ Wv + bv
        return carry, (kk[0], vv[0])

    _, (kks, vvs) = jax.lax.scan(proj, jnp.float32(0), xs)

    wqf = Wq.transpose(1, 0, 2).reshape(H, E * IKS)
    wif = Wi.transpose(1, 0, 2).reshape(IVS, E * G4)
    whf = Wh.transpose(1, 0, 2).reshape(H, E * G4)
    wq2f = Wq2.transpose(1, 0, 2).reshape(H, E * NCH * CKS)
    wk2f = Wk2.transpose(1, 0, 2).reshape(H, E * NCH * CKS)
    wv2f = Wv2.transpose(1, 0, 2).reshape(H, E * NCH * CVS)
    wqkv = jnp.concatenate([wq2f, wk2f, wv2f], axis=1)

    n2 = T // CHUNK
    rw, idxo = pl.pallas_call(
        _stage2,
        grid=(n2,),
        in_specs=[
            pl.BlockSpec((CHUNK, 2, IKS), lambda i: (i, 0, 0)),
            pl.BlockSpec((CHUNK, 2, IVS), lambda i: (i, 0, 0)),
            pl.BlockSpec((1, E, H), lambda i: (0, 0, 0)),
            pl.BlockSpec((1, E, H), lambda i: (0, 0, 0)),
            pl.BlockSpec((H, E * IKS), lambda i: (0, 0)),
            pl.BlockSpec((IVS, E * G4), lambda i: (0, 0)),
            pl.BlockSpec((H, E * G4), lambda i: (0, 0)),
            pl.BlockSpec((H, E * NCH * (2 * CKS + CVS)), lambda i: (0, 0)),
            pl.BlockSpec((E, NCH * CVS, CVS), lambda i: (0, 0, 0)),
        ],
        out_specs=[
            pl.BlockSpec((CHUNK, 128), lambda i: (i, 0)),
            pl.BlockSpec((CHUNK, 128), lambda i: (i, 0)),
        ],
        out_shape=[
            jax.ShapeDtypeStruct((T, 128), jnp.float32),
            jax.ShapeDtypeStruct((T, 128), jnp.int32),
        ],
        scratch_shapes=[
            pltpu.VMEM((E, H), jnp.float32),
            pltpu.VMEM((E, H), jnp.float32),
        ],
    )(kks, vvs, h0, c0, wqf, wif, whf, wqkv, Wout)

    routing_weights = rw[:, :K].reshape(-1)
    selected_experts = idxo[:, :K].reshape(-1)
    return routing_weights, selected_experts
